# baseline (speedup 1.0000x reference)
import functools

import jax
import jax.numpy as jnp
from jax import lax
from jax.experimental import pallas as pl
from jax.experimental.pallas import tpu as pltpu
from jax.experimental.pallas import tpu_sc as plsc

NC = 2
NS = 16
CH = 128


def _sc_segment_sum(src3, dst3, h, zeros, n_acc, rpt, nch, hd):
    mesh = plsc.VectorSubcoreMesh(
        core_axis_name="c", subcore_axis_name="s", num_cores=NC, num_subcores=NS
    )

    def body(src_hbm, dst_hbm, h_hbm, zeros_hbm, out_hbm,
             src_v, dst_v, buf0, buf1, buf2, buf3, acc,
             sg0, sg1, sg2, sg3, ss0, ss1, ss2, ss3):
        c = lax.axis_index("c")
        s = lax.axis_index("s")
        my_h = h_hbm.at[c]
        bufs = (buf0, buf1, buf2, buf3)
        sg = (sg0, sg1, sg2, sg3)
        ss = (ss0, ss1, ss2, ss3)

        def gather(j, b):
            pltpu.async_copy(my_h.at[src_v.at[j]], bufs[b], sg[b])

        def gather_wait(j, b):
            pltpu.make_async_copy(my_h.at[src_v.at[j]], bufs[b], sg[b]).wait()

        def scatter(j, b):
            pltpu.async_copy(bufs[b], acc.at[dst_v.at[j]], ss[b], add=True)

        def scatter_wait(j, b):
            pltpu.make_async_copy(bufs[b], acc.at[dst_v.at[j]], ss[b]).wait()

        src_stage = pltpu.async_copy(src_hbm.at[s], src_v, ss[0])
        dst_stage = pltpu.async_copy(dst_hbm.at[s], dst_v, ss[1])
        src_stage.wait()

        gather(0, 0)
        gather(1, 1)

        pltpu.sync_copy(zeros_hbm, acc.at[pl.ds(s * rpt, rpt)])
        dst_stage.wait()
        plsc.subcore_barrier()

        def steady(k, carry):
            for b in range(2):
                i = 2 * k + b
                gather_wait(i, b)
                pltpu.sync_copy(bufs[b], acc.at[dst_v.at[i]], add=True)
                gather(i + 2, b)
            return carry

        lax.fori_loop(0, nch // 2 - 1, steady, 0)

        for b in range(2):
            i = nch - 2 + b
            gather_wait(i, b)
            pltpu.sync_copy(bufs[b], acc.at[dst_v.at[i]], add=True)

        plsc.subcore_barrier()
        pltpu.sync_copy(acc.at[pl.ds(s * rpt, rpt)],
                        out_hbm.at[c, pl.ds(s * rpt, rpt)])

    fn = pl.kernel(
        body,
        out_type=jax.ShapeDtypeStruct((NC, n_acc, hd), jnp.float32),
        mesh=mesh,
        scratch_types=[
            pltpu.VMEM((nch, CH), jnp.int32),
            pltpu.VMEM((nch, CH), jnp.int32),
            pltpu.VMEM((CH, hd), jnp.float32),
            pltpu.VMEM((CH, hd), jnp.float32),
            pltpu.VMEM((CH, hd), jnp.float32),
            pltpu.VMEM((CH, hd), jnp.float32),
            pltpu.VMEM_SHARED((n_acc, hd), jnp.float32),
            pltpu.SemaphoreType.DMA,
            pltpu.SemaphoreType.DMA,
            pltpu.SemaphoreType.DMA,
            pltpu.SemaphoreType.DMA,
            pltpu.SemaphoreType.DMA,
            pltpu.SemaphoreType.DMA,
            pltpu.SemaphoreType.DMA,
            pltpu.SemaphoreType.DMA,
        ],
        compiler_params=pltpu.CompilerParams(use_tc_tiling_on_sc=False),
    )
    return fn(src3, dst3, h, zeros)


def _tc_self_body(h_ref, ws_ref, bs_ref, sh_ref):
    sh_ref[...] = jnp.dot(h_ref[...], ws_ref[...],
                          preferred_element_type=jnp.float32) + bs_ref[...]


def _tc_body(pad_cnt, sh_ref, p_ref, h0_ref, wn_ref, bn_ref, g_ref, be_ref,
             out_ref):
    p = jnp.concatenate([p_ref[0], p_ref[1]], axis=1)
    if pad_cnt:
        row0 = (lax.broadcasted_iota(jnp.int32, (p.shape[0], 1), 0) == 0) & (
            pl.program_id(0) == 0)
        p = p - jnp.where(row0, jnp.float32(pad_cnt), 0.0) * h0_ref[...]
    sh = sh_ref[...]
    nh = jnp.dot(p, wn_ref[...], preferred_element_type=jnp.float32) + bn_ref[...]
    cat = jnp.concatenate([sh, nh], axis=1)
    mu = jnp.mean(cat, axis=1, keepdims=True)
    var = jnp.mean((cat - mu) * (cat - mu), axis=1, keepdims=True)
    out_ref[...] = (cat - mu) * lax.rsqrt(var + 1e-5) * g_ref[...] + be_ref[...]


def kernel(edge_index, h, W_self, b_self, W_neigh, b_neigh, gamma, beta):
    n, d = h.shape
    o = W_self.shape[1]
    e = edge_index.shape[1]
    hd = d // NC

    nch = -(-e // (NS * CH))
    nch += nch % 2
    e_pad = NS * CH * nch
    pad_cnt = e_pad - e
    ei = jnp.pad(edge_index, ((0, 0), (0, pad_cnt)))
    src3 = ei[1].reshape(NS, nch, CH)
    dst3 = ei[0].reshape(NS, nch, CH)
    h2 = jnp.transpose(h.reshape(n, NC, hd), (1, 0, 2))

    rpt = 8 * (-(-(n + 1) // (NS * 8)))
    n_acc = NS * rpt
    zeros = jnp.zeros((rpt, hd), jnp.float32)

    partial = _sc_segment_sum(src3, dst3, h2, zeros, n_acc, rpt, nch, hd)

    blk = 2000
    grid = n // blk
    sh = pl.pallas_call(
        _tc_self_body,
        grid=(grid,),
        in_specs=[
            pl.BlockSpec((blk, d), lambda i: (i, 0)),
            pl.BlockSpec((d, o), lambda i: (0, 0)),
            pl.BlockSpec((1, o), lambda i: (0, 0)),
        ],
        out_specs=pl.BlockSpec((blk, o), lambda i: (i, 0)),
        out_shape=jax.ShapeDtypeStruct((n, o), jnp.float32),
    )(h, W_self, b_self.reshape(1, o))

    out = pl.pallas_call(
        functools.partial(_tc_body, pad_cnt),
        grid=(grid,),
        in_specs=[
            pl.BlockSpec((blk, o), lambda i: (i, 0)),
            pl.BlockSpec((NC, blk, hd), lambda i: (0, i, 0)),
            pl.BlockSpec((1, d), lambda i: (0, 0)),
            pl.BlockSpec((d, o), lambda i: (0, 0)),
            pl.BlockSpec((1, o), lambda i: (0, 0)),
            pl.BlockSpec((1, 2 * o), lambda i: (0, 0)),
            pl.BlockSpec((1, 2 * o), lambda i: (0, 0)),
        ],
        out_specs=pl.BlockSpec((blk, 2 * o), lambda i: (i, 0)),
        out_shape=jax.ShapeDtypeStruct((n, 2 * o), jnp.float32),
    )(sh, partial, h[0:1], W_neigh, b_neigh.reshape(1, o),
      gamma.reshape(1, 2 * o), beta.reshape(1, 2 * o))
    return out


# speedup vs baseline: 1.0107x; 1.0107x over previous
import functools

import jax
import jax.numpy as jnp
from jax import lax
from jax.experimental import pallas as pl
from jax.experimental.pallas import tpu as pltpu
from jax.experimental.pallas import tpu_sc as plsc

NC = 2
NS = 16
CH = 128


def _sc_segment_sum(src3, dst3, h, zeros, n_acc, rpt, nch, hd):
    mesh = plsc.VectorSubcoreMesh(
        core_axis_name="c", subcore_axis_name="s", num_cores=NC, num_subcores=NS
    )

    def body(src_hbm, dst_hbm, h_hbm, zeros_hbm, out_hbm,
             src_v, dst_v, buf0, buf1, acc, sg0, sg1, st0, st1):
        c = lax.axis_index("c")
        s = lax.axis_index("s")
        my_h = h_hbm.at[c]
        bufs = (buf0, buf1)
        sg = (sg0, sg1)

        def gather(j, b):
            pltpu.async_copy(my_h.at[src_v.at[j]], bufs[b], sg[b])

        def gather_wait(j, b):
            pltpu.make_async_copy(my_h.at[src_v.at[j]], bufs[b], sg[b]).wait()

        src_stage = pltpu.async_copy(src_hbm.at[s], src_v, st0)
        dst_stage = pltpu.async_copy(dst_hbm.at[s], dst_v, st1)
        src_stage.wait()

        gather(0, 0)
        gather(1, 1)

        pltpu.sync_copy(zeros_hbm, acc.at[pl.ds(s * rpt, rpt)])
        dst_stage.wait()
        plsc.subcore_barrier()

        def steady(k, carry):
            for b in range(2):
                i = 2 * k + b
                gather_wait(i, b)
                pltpu.sync_copy(bufs[b], acc.at[dst_v.at[i]], add=True)
                gather(i + 2, b)
            return carry

        lax.fori_loop(0, nch // 2 - 1, steady, 0)

        for b in range(2):
            i = nch - 2 + b
            gather_wait(i, b)
            pltpu.sync_copy(bufs[b], acc.at[dst_v.at[i]], add=True)

        plsc.subcore_barrier()
        pltpu.sync_copy(acc.at[pl.ds(s * rpt, rpt)],
                        out_hbm.at[c, pl.ds(s * rpt, rpt)])

    fn = pl.kernel(
        body,
        out_type=jax.ShapeDtypeStruct((NC, n_acc, hd), jnp.float32),
        mesh=mesh,
        scratch_types=[
            pltpu.VMEM((nch, CH), jnp.int32),
            pltpu.VMEM((nch, CH), jnp.int32),
            pltpu.VMEM((CH, hd), jnp.float32),
            pltpu.VMEM((CH, hd), jnp.float32),
            pltpu.VMEM_SHARED((n_acc, hd), jnp.float32),
            pltpu.SemaphoreType.DMA,
            pltpu.SemaphoreType.DMA,
            pltpu.SemaphoreType.DMA,
            pltpu.SemaphoreType.DMA,
        ],
        compiler_params=pltpu.CompilerParams(use_tc_tiling_on_sc=False),
    )
    return fn(src3, dst3, h, zeros)


def _tc_body(pad_cnt, h_ref, p_ref, ws_ref, wn_ref, bs_ref, bn_ref, g_ref, be_ref,
             out_ref):
    x = h_ref[...]
    p = jnp.concatenate([p_ref[0], p_ref[1]], axis=1)
    if pad_cnt:
        row0 = (lax.broadcasted_iota(jnp.int32, (p.shape[0], 1), 0) == 0) & (
            pl.program_id(0) == 0)
        p = p - jnp.where(row0, jnp.float32(pad_cnt), 0.0) * x
    sh = jnp.dot(x, ws_ref[...], preferred_element_type=jnp.float32) + bs_ref[...]
    nh = jnp.dot(p, wn_ref[...], preferred_element_type=jnp.float32) + bn_ref[...]
    cat = jnp.concatenate([sh, nh], axis=1)
    mu = jnp.mean(cat, axis=1, keepdims=True)
    var = jnp.mean((cat - mu) * (cat - mu), axis=1, keepdims=True)
    out_ref[...] = (cat - mu) * lax.rsqrt(var + 1e-5) * g_ref[...] + be_ref[...]


def kernel(edge_index, h, W_self, b_self, W_neigh, b_neigh, gamma, beta):
    n, d = h.shape
    o = W_self.shape[1]
    e = edge_index.shape[1]
    hd = d // NC

    nch = -(-e // (NS * CH))
    nch += nch % 2
    e_pad = NS * CH * nch
    pad_cnt = e_pad - e
    ei = jnp.pad(edge_index, ((0, 0), (0, pad_cnt)))
    src3 = ei[1].reshape(NS, nch, CH)
    dst3 = ei[0].reshape(NS, nch, CH)
    h2 = jnp.transpose(h.reshape(n, NC, hd), (1, 0, 2))

    rpt = 8 * (-(-(n + 1) // (NS * 8)))
    n_acc = NS * rpt
    zeros = jnp.zeros((rpt, hd), jnp.float32)

    partial = _sc_segment_sum(src3, dst3, h2, zeros, n_acc, rpt, nch, hd)

    blk = 2000
    grid = n // blk
    out = pl.pallas_call(
        functools.partial(_tc_body, pad_cnt),
        grid=(grid,),
        in_specs=[
            pl.BlockSpec((blk, d), lambda i: (i, 0)),
            pl.BlockSpec((NC, blk, hd), lambda i: (0, i, 0)),
            pl.BlockSpec((d, o), lambda i: (0, 0)),
            pl.BlockSpec((d, o), lambda i: (0, 0)),
            pl.BlockSpec((1, o), lambda i: (0, 0)),
            pl.BlockSpec((1, o), lambda i: (0, 0)),
            pl.BlockSpec((1, 2 * o), lambda i: (0, 0)),
            pl.BlockSpec((1, 2 * o), lambda i: (0, 0)),
        ],
        out_specs=pl.BlockSpec((blk, 2 * o), lambda i: (i, 0)),
        out_shape=jax.ShapeDtypeStruct((n, 2 * o), jnp.float32),
    )(h, partial, W_self, W_neigh, b_self.reshape(1, o), b_neigh.reshape(1, o),
      gamma.reshape(1, 2 * o), beta.reshape(1, 2 * o))
    return out


# speedup vs baseline: 1.0131x; 1.0024x over previous
import functools

import jax
import jax.numpy as jnp
from jax import lax
from jax.experimental import pallas as pl
from jax.experimental.pallas import tpu as pltpu
from jax.experimental.pallas import tpu_sc as plsc

NC = 2
NS = 16
CH = 128


def _sc_segment_sum(src3, dst3, h, zeros, n_acc, rpt, nch, hd):
    mesh = plsc.VectorSubcoreMesh(
        core_axis_name="c", subcore_axis_name="s", num_cores=NC, num_subcores=NS
    )

    def body(src_hbm, dst_hbm, h_hbm, zeros_hbm, out_hbm,
             src_v, dst_v, buf0, buf1, acc, sg0, sg1, st0, st1):
        c = lax.axis_index("c")
        s = lax.axis_index("s")
        my_h = h_hbm.at[c]
        bufs = (buf0, buf1)
        sg = (sg0, sg1)

        def gather(j, b):
            pltpu.async_copy(my_h.at[src_v.at[j]], bufs[b], sg[b])

        def gather_wait(j, b):
            pltpu.make_async_copy(my_h.at[src_v.at[j]], bufs[b], sg[b]).wait()

        src_stage = pltpu.async_copy(src_hbm.at[s], src_v, st0)
        dst_stage = pltpu.async_copy(dst_hbm.at[s], dst_v, st1)
        src_stage.wait()

        gather(0, 0)
        gather(1, 1)

        pltpu.sync_copy(zeros_hbm, acc.at[pl.ds(s * rpt, rpt)])
        dst_stage.wait()
        plsc.subcore_barrier()

        def steady(k, carry):
            for b in range(2):
                i = 2 * k + b
                gather_wait(i, b)
                pltpu.sync_copy(bufs[b], acc.at[dst_v.at[i]], add=True)
                gather(i + 2, b)
            return carry

        lax.fori_loop(0, nch // 2 - 1, steady, 0)

        for b in range(2):
            i = nch - 2 + b
            gather_wait(i, b)
            pltpu.sync_copy(bufs[b], acc.at[dst_v.at[i]], add=True)

        plsc.subcore_barrier()
        pltpu.sync_copy(acc.at[pl.ds(s * rpt, rpt)],
                        out_hbm.at[c, pl.ds(s * rpt, rpt)])

    fn = pl.kernel(
        body,
        out_type=jax.ShapeDtypeStruct((NC, n_acc, hd), jnp.float32),
        mesh=mesh,
        scratch_types=[
            pltpu.VMEM((nch, CH), jnp.int32),
            pltpu.VMEM((nch, CH), jnp.int32),
            pltpu.VMEM((CH, hd), jnp.float32),
            pltpu.VMEM((CH, hd), jnp.float32),
            pltpu.VMEM_SHARED((n_acc, hd), jnp.float32),
            pltpu.SemaphoreType.DMA,
            pltpu.SemaphoreType.DMA,
            pltpu.SemaphoreType.DMA,
            pltpu.SemaphoreType.DMA,
        ],
        compiler_params=pltpu.CompilerParams(use_tc_tiling_on_sc=False),
    )
    return fn(src3, dst3, h, zeros)


def _tc_body(pad_cnt, h_ref, p_ref, ws_ref, wn_ref, bs_ref, bn_ref, g_ref, be_ref,
             out_ref):
    x = h_ref[...]
    hd = p_ref.shape[2] // 2
    halves = []
    for c in range(p_ref.shape[0]):
        xc = p_ref[c]
        halves.append(
            jnp.stack([xc[:, :hd], xc[:, hd:]], axis=1).reshape(-1, hd))
    p = jnp.concatenate(halves, axis=1)
    if pad_cnt:
        row0 = (lax.broadcasted_iota(jnp.int32, (p.shape[0], 1), 0) == 0) & (
            pl.program_id(0) == 0)
        p = p - jnp.where(row0, jnp.float32(pad_cnt), 0.0) * x
    sh = jnp.dot(x, ws_ref[...], preferred_element_type=jnp.float32) + bs_ref[...]
    nh = jnp.dot(p, wn_ref[...], preferred_element_type=jnp.float32) + bn_ref[...]
    cat = jnp.concatenate([sh, nh], axis=1)
    mu = jnp.mean(cat, axis=1, keepdims=True)
    var = jnp.mean((cat - mu) * (cat - mu), axis=1, keepdims=True)
    out_ref[...] = (cat - mu) * lax.rsqrt(var + 1e-5) * g_ref[...] + be_ref[...]


def kernel(edge_index, h, W_self, b_self, W_neigh, b_neigh, gamma, beta):
    n, d = h.shape
    o = W_self.shape[1]
    e = edge_index.shape[1]
    hd = d // NC

    nch = -(-e // (NS * CH))
    nch += nch % 2
    e_pad = NS * CH * nch
    pad_cnt = e_pad - e
    ei = jnp.pad(edge_index, ((0, 0), (0, pad_cnt)))
    src3 = ei[1].reshape(NS, nch, CH)
    dst3 = ei[0].reshape(NS, nch, CH)
    h2 = jnp.transpose(h.reshape(n, NC, hd), (1, 0, 2))

    rpt = 8 * (-(-(n + 1) // (NS * 8)))
    n_acc = NS * rpt
    zeros = jnp.zeros((rpt, hd), jnp.float32)

    partial = _sc_segment_sum(src3, dst3, h2, zeros, n_acc, rpt, nch, hd)
    partial = partial.reshape(NC, n_acc // 2, 2 * hd)

    blk = 2000
    grid = n // blk
    out = pl.pallas_call(
        functools.partial(_tc_body, pad_cnt),
        grid=(grid,),
        in_specs=[
            pl.BlockSpec((blk, d), lambda i: (i, 0)),
            pl.BlockSpec((NC, blk // 2, 2 * hd), lambda i: (0, i, 0)),
            pl.BlockSpec((d, o), lambda i: (0, 0)),
            pl.BlockSpec((d, o), lambda i: (0, 0)),
            pl.BlockSpec((1, o), lambda i: (0, 0)),
            pl.BlockSpec((1, o), lambda i: (0, 0)),
            pl.BlockSpec((1, 2 * o), lambda i: (0, 0)),
            pl.BlockSpec((1, 2 * o), lambda i: (0, 0)),
        ],
        out_specs=pl.BlockSpec((blk, 2 * o), lambda i: (i, 0)),
        out_shape=jax.ShapeDtypeStruct((n, 2 * o), jnp.float32),
    )(h, partial, W_self, W_neigh, b_self.reshape(1, o), b_neigh.reshape(1, o),
      gamma.reshape(1, 2 * o), beta.reshape(1, 2 * o))
    return out
